# Initial kernel scaffold; baseline (speedup 1.0000x reference)
import jax
import jax.numpy as jnp
from jax.experimental import pallas as pl


def kernel(hidden_states, weight, e_score_correction_bias):
    raise NotImplementedError("write your pallas kernel here")


# speedup vs baseline: 1.4580x; 1.4580x over previous
import functools

import jax
import jax.numpy as jnp
from jax.experimental import pallas as pl

_HID = 2048
_NE = 64
_K = 8


def _router_block(x_ref, wt_ref, b_ref, idx_ref, wgt_ref):
    x = x_ref[...]
    wt = wt_ref[...]
    logits = jnp.dot(x, wt, preferred_element_type=jnp.float32)
    scores = jax.nn.sigmoid(logits)
    biased = scores + b_ref[...]

    col = jax.lax.broadcasted_iota(jnp.int32, biased.shape, 1)
    cur = biased
    picked_i = []
    picked_w = []
    for _ in range(_K):
        m = jnp.max(cur, axis=1, keepdims=True)
        eq = cur == m
        idx = jnp.min(jnp.where(eq, col, _NE), axis=1, keepdims=True)
        onehot = col == idx
        w = jnp.sum(jnp.where(onehot, scores, 0.0), axis=1, keepdims=True)
        picked_i.append(idx)
        picked_w.append(w)
        cur = jnp.where(onehot, -jnp.inf, cur)

    idxs = jnp.concatenate(picked_i, axis=1)
    wgts = jnp.concatenate(picked_w, axis=1)
    denom = jnp.sum(wgts, axis=1, keepdims=True) + 1e-20
    idx_ref[...] = idxs
    wgt_ref[...] = wgts / denom


@jax.jit
def kernel(hidden_states, weight, e_score_correction_bias):
    x = hidden_states.reshape(-1, _HID).astype(jnp.float32)
    t = x.shape[0]
    bt = 1024
    wt = weight.astype(jnp.float32).T
    bias = e_score_correction_bias.astype(jnp.float32).reshape(1, _NE)

    grid = (t // bt,)
    out = pl.pallas_call(
        _router_block,
        grid=grid,
        in_specs=[
            pl.BlockSpec((bt, _HID), lambda i: (i, 0)),
            pl.BlockSpec((_HID, _NE), lambda i: (0, 0)),
            pl.BlockSpec((1, _NE), lambda i: (0, 0)),
        ],
        out_specs=[
            pl.BlockSpec((bt, _K), lambda i: (i, 0)),
            pl.BlockSpec((bt, _K), lambda i: (i, 0)),
        ],
        out_shape=[
            jax.ShapeDtypeStruct((t, _K), jnp.int32),
            jax.ShapeDtypeStruct((t, _K), jnp.float32),
        ],
    )(x, wt, bias)
    return out[0], out[1]


# speedup vs baseline: 2.3305x; 1.5984x over previous
import functools

import jax
import jax.numpy as jnp
from jax.experimental import pallas as pl

_HID = 2048
_NE = 64
_K = 8


def _router_block(x_ref, wt_ref, b_ref, idx_ref, wgt_ref):
    x = x_ref[...]
    wt = wt_ref[...]
    logits = jnp.dot(x, wt, preferred_element_type=jnp.float32)
    logits_t = logits.T
    scores = jax.nn.sigmoid(logits_t)
    biased = scores + b_ref[...]

    row = jax.lax.broadcasted_iota(jnp.int32, biased.shape, 0)
    cur = biased
    picked_i = []
    picked_w = []
    for _ in range(_K):
        m = jnp.max(cur, axis=0, keepdims=True)
        eq = cur == m
        idx = jnp.min(jnp.where(eq, row, _NE), axis=0, keepdims=True)
        onehot = row == idx
        w = jnp.sum(jnp.where(onehot, scores, 0.0), axis=0, keepdims=True)
        picked_i.append(idx)
        picked_w.append(w)
        cur = jnp.where(onehot, -jnp.inf, cur)

    idx_t = jnp.concatenate(picked_i, axis=0)
    wgt_t = jnp.concatenate(picked_w, axis=0)
    denom = jnp.sum(wgt_t, axis=0, keepdims=True) + 1e-20
    idx_ref[...] = idx_t.T
    wgt_ref[...] = (wgt_t / denom).T


@jax.jit
def kernel(hidden_states, weight, e_score_correction_bias):
    x = hidden_states.reshape(-1, _HID).astype(jnp.float32)
    t = x.shape[0]
    bt = 1024
    wt = weight.astype(jnp.float32).T
    bias = e_score_correction_bias.astype(jnp.float32).reshape(_NE, 1)

    grid = (t // bt,)
    out = pl.pallas_call(
        _router_block,
        grid=grid,
        in_specs=[
            pl.BlockSpec((bt, _HID), lambda i: (i, 0)),
            pl.BlockSpec((_HID, _NE), lambda i: (0, 0)),
            pl.BlockSpec((_NE, 1), lambda i: (0, 0)),
        ],
        out_specs=[
            pl.BlockSpec((bt, _K), lambda i: (i, 0)),
            pl.BlockSpec((bt, _K), lambda i: (i, 0)),
        ],
        out_shape=[
            jax.ShapeDtypeStruct((t, _K), jnp.int32),
            jax.ShapeDtypeStruct((t, _K), jnp.float32),
        ],
    )(x, wt, bias)
    return out[0], out[1]


# speedup vs baseline: 2.4689x; 1.0594x over previous
import functools

import jax
import jax.numpy as jnp
from jax.experimental import pallas as pl

_HID = 2048
_NE = 64
_K = 8


def _router_block(x_ref, wt_ref, b_ref, idx_ref, wgt_ref):
    x = x_ref[...]
    wt = wt_ref[...]
    logits = jnp.dot(x, wt, preferred_element_type=jnp.float32)
    logits_t = logits.T
    scores = jax.nn.sigmoid(logits_t)
    biased = scores + b_ref[...]

    row = jax.lax.broadcasted_iota(jnp.int32, biased.shape, 0)
    cur = biased
    picked_i = []
    picked_w = []
    for _ in range(_K):
        m = jnp.max(cur, axis=0, keepdims=True)
        eq = cur == m
        idx = jnp.min(jnp.where(eq, row, _NE), axis=0, keepdims=True)
        onehot = row == idx
        w = jnp.sum(jnp.where(onehot, scores, 0.0), axis=0, keepdims=True)
        picked_i.append(idx)
        picked_w.append(w)
        cur = jnp.where(onehot, -jnp.inf, cur)

    idx_t = jnp.concatenate(picked_i, axis=0)
    wgt_t = jnp.concatenate(picked_w, axis=0)
    denom = jnp.sum(wgt_t, axis=0, keepdims=True) + 1e-20
    idx_ref[...] = idx_t.T
    wgt_ref[...] = (wgt_t / denom).T


@jax.jit
def kernel(hidden_states, weight, e_score_correction_bias):
    x = hidden_states.reshape(-1, _HID).astype(jnp.float32)
    t = x.shape[0]
    bt = 2048
    wt = weight.astype(jnp.float32).T
    bias = e_score_correction_bias.astype(jnp.float32).reshape(_NE, 1)

    grid = (t // bt,)
    out = pl.pallas_call(
        _router_block,
        grid=grid,
        in_specs=[
            pl.BlockSpec((bt, _HID), lambda i: (i, 0)),
            pl.BlockSpec((_HID, _NE), lambda i: (0, 0)),
            pl.BlockSpec((_NE, 1), lambda i: (0, 0)),
        ],
        out_specs=[
            pl.BlockSpec((bt, _K), lambda i: (i, 0)),
            pl.BlockSpec((bt, _K), lambda i: (i, 0)),
        ],
        out_shape=[
            jax.ShapeDtypeStruct((t, _K), jnp.int32),
            jax.ShapeDtypeStruct((t, _K), jnp.float32),
        ],
    )(x, wt, bias)
    return out[0], out[1]
